# Initial kernel scaffold; baseline (speedup 1.0000x reference)
import jax
import jax.numpy as jnp
from jax.experimental import pallas as pl


def kernel(x, edge_index, W1, b1, W2, b2, W3, b3):
    raise NotImplementedError("write your pallas kernel here")


# speedup vs baseline: 6.1207x; 6.1207x over previous
import functools

import jax
import jax.numpy as jnp
from jax import lax
from jax.experimental import pallas as pl
from jax.experimental.pallas import tpu as pltpu
from jax.experimental.pallas import tpu_sc as plsc

N_NODES = 10000
D = 128
NC = 2
NS = 16
CHUNK = 128
N_PAD = 10240
HALF = N_PAD // NC
ACC_ROWS = HALF + 8
TRASH = HALF
ROWS_PER_TILE = HALF // NS
PAD_IDX = N_PAD - 1
BLK = 256
N_BLOCKS = N_PAD // BLK


def _sc_agg_body(nchunk, table, srcs, dsts, zinit, out,
                 src_v, dst_v, rows0, rows1, sem0, sem1, acc):
    c = lax.axis_index("c")
    s = lax.axis_index("s")
    r0 = s * ROWS_PER_TILE
    pltpu.sync_copy(zinit, acc.at[pl.ds(r0, ROWS_PER_TILE)])
    pltpu.sync_copy(srcs.at[s], src_v)
    pltpu.sync_copy(dsts.at[c, s], dst_v)
    plsc.subcore_barrier()

    def body(j, carry):
        cp_a = pltpu.async_copy(table.at[src_v.at[2 * j]], rows0, sem0)
        cp_b = pltpu.async_copy(table.at[src_v.at[2 * j + 1]], rows1, sem1)
        cp_a.wait()
        pltpu.sync_copy(rows0, acc.at[dst_v.at[2 * j]], add=True)
        cp_b.wait()
        pltpu.sync_copy(rows1, acc.at[dst_v.at[2 * j + 1]], add=True)
        return carry

    lax.fori_loop(0, nchunk // 2, body, 0)
    plsc.subcore_barrier()
    pltpu.sync_copy(acc.at[pl.ds(r0, ROWS_PER_TILE)],
                    out.at[c, pl.ds(r0, ROWS_PER_TILE)])


def _sc_aggregate(table, srcs, dsts, zinit, d, nchunk):
    mesh = plsc.VectorSubcoreMesh(core_axis_name="c", subcore_axis_name="s",
                                  num_cores=NC, num_subcores=NS)
    kern = pl.kernel(
        functools.partial(_sc_agg_body, nchunk),
        out_type=jax.ShapeDtypeStruct((NC, HALF, d), jnp.float32),
        mesh=mesh,
        scratch_types=[
            pltpu.VMEM((nchunk, CHUNK), jnp.int32),
            pltpu.VMEM((nchunk, CHUNK), jnp.int32),
            pltpu.VMEM((CHUNK, d), jnp.float32),
            pltpu.VMEM((CHUNK, d), jnp.float32),
            pltpu.SemaphoreType.DMA,
            pltpu.SemaphoreType.DMA,
            pltpu.VMEM_SHARED((ACC_ROWS, d), jnp.float32),
        ],
        compiler_params=pltpu.CompilerParams(use_tc_tiling_on_sc=False),
        name=f"gcn_sc_agg_d{d}",
    )
    return kern(table, srcs, dsts, zinit)


def _tc_first_body(x_ref, w_ref, degp_ref, hs_ref, dinv_ref):
    i = pl.program_id(0)
    deg = jnp.sum(degp_ref[...], axis=1) + 1.0
    row = i * BLK + lax.broadcasted_iota(jnp.int32, (BLK,), 0)
    dinv = jnp.where(row < N_NODES, lax.rsqrt(deg), 0.0)
    dinv_b = jnp.broadcast_to(dinv[:, None], (BLK, D))
    dinv_ref[...] = dinv_b
    h = jnp.dot(x_ref[...], w_ref[...], preferred_element_type=jnp.float32)
    hs_ref[...] = h * dinv_b


def _tc_first(x_pad, w1, degs):
    return pl.pallas_call(
        _tc_first_body,
        grid=(N_BLOCKS,),
        in_specs=[
            pl.BlockSpec((BLK, D), lambda i: (i, 0)),
            pl.BlockSpec((D, D), lambda i: (0, 0)),
            pl.BlockSpec((BLK, 16), lambda i: (i, 0)),
        ],
        out_specs=[
            pl.BlockSpec((BLK, D), lambda i: (i, 0)),
            pl.BlockSpec((BLK, D), lambda i: (i, 0)),
        ],
        out_shape=[
            jax.ShapeDtypeStruct((N_PAD, D), jnp.float32),
            jax.ShapeDtypeStruct((N_PAD, D), jnp.float32),
        ],
        name="gcn_tc_first",
    )(x_pad, w1, degs)


def _tc_mid_body(agg_ref, hs_ref, dinv_ref, b_ref, w_ref, o_ref):
    tot = agg_ref[...] + hs_ref[...]
    act = jnp.maximum(tot * dinv_ref[...] + b_ref[...], 0.0)
    o_ref[...] = jnp.dot(act, w_ref[...],
                         preferred_element_type=jnp.float32) * dinv_ref[...]


def _tc_mid(agg, hs, dinv_b, b, w_next):
    return pl.pallas_call(
        _tc_mid_body,
        grid=(N_BLOCKS,),
        in_specs=[
            pl.BlockSpec((BLK, D), lambda i: (i, 0)),
            pl.BlockSpec((BLK, D), lambda i: (i, 0)),
            pl.BlockSpec((BLK, D), lambda i: (i, 0)),
            pl.BlockSpec((1, D), lambda i: (0, 0)),
            pl.BlockSpec((D, D), lambda i: (0, 0)),
        ],
        out_specs=pl.BlockSpec((BLK, D), lambda i: (i, 0)),
        out_shape=jax.ShapeDtypeStruct((N_PAD, D), jnp.float32),
        name="gcn_tc_mid",
    )(agg, hs, dinv_b, b.reshape(1, D), w_next)


def _tc_final_body(agg_ref, hs_ref, dinv_ref, b_ref, o_ref):
    tot = agg_ref[...] + hs_ref[...]
    o_ref[...] = tot * dinv_ref[...] + b_ref[...]


def _tc_final(agg, hs, dinv_b, b):
    return pl.pallas_call(
        _tc_final_body,
        grid=(N_BLOCKS,),
        in_specs=[
            pl.BlockSpec((BLK, D), lambda i: (i, 0)),
            pl.BlockSpec((BLK, D), lambda i: (i, 0)),
            pl.BlockSpec((BLK, D), lambda i: (i, 0)),
            pl.BlockSpec((1, D), lambda i: (0, 0)),
        ],
        out_specs=pl.BlockSpec((BLK, D), lambda i: (i, 0)),
        out_shape=jax.ShapeDtypeStruct((N_PAD, D), jnp.float32),
        name="gcn_tc_final",
    )(agg, hs, dinv_b, b.reshape(1, D))


def kernel(x, edge_index, W1, b1, W2, b2, W3, b3):
    e = edge_index.shape[1]
    nchunk = -(-e // (NS * CHUNK))
    if nchunk % 2:
        nchunk += 1
    e_pad = NS * nchunk * CHUNK
    src = edge_index[0].astype(jnp.int32)
    dst = edge_index[1].astype(jnp.int32)
    fill = jnp.full((e_pad - e,), PAD_IDX, jnp.int32)
    src = jnp.concatenate([src, fill])
    dst = jnp.concatenate([dst, fill])
    srcs = src.reshape(NS, nchunk, CHUNK)
    dst_loc = dst[None, :] - jnp.array([0, HALF], jnp.int32)[:, None]
    dsts = jnp.where((dst_loc >= 0) & (dst_loc < HALF), dst_loc, TRASH)
    dsts = dsts.reshape(NC, NS, nchunk, CHUNK)

    x_pad = jnp.pad(x, ((0, N_PAD - N_NODES), (0, 0)))
    e0_table = jnp.zeros((N_PAD, 16), jnp.float32).at[:N_NODES, 0].set(1.0)
    z16 = jnp.zeros((ROWS_PER_TILE, 16), jnp.float32)
    z128 = jnp.zeros((ROWS_PER_TILE, D), jnp.float32)

    degs = _sc_aggregate(e0_table, srcs, dsts, z16, 16, nchunk)
    hs1, dinv_b = _tc_first(x_pad, W1, degs.reshape(N_PAD, 16))
    agg = _sc_aggregate(hs1, srcs, dsts, z128, D, nchunk).reshape(N_PAD, D)
    hs2 = _tc_mid(agg, hs1, dinv_b, b1, W2)
    agg = _sc_aggregate(hs2, srcs, dsts, z128, D, nchunk).reshape(N_PAD, D)
    hs3 = _tc_mid(agg, hs2, dinv_b, b2, W3)
    agg = _sc_aggregate(hs3, srcs, dsts, z128, D, nchunk).reshape(N_PAD, D)
    out = _tc_final(agg, hs3, dinv_b, b3)
    return out[:N_NODES]


# speedup vs baseline: 6.4996x; 1.0619x over previous
import functools

import jax
import jax.numpy as jnp
from jax import lax
from jax.experimental import pallas as pl
from jax.experimental.pallas import tpu as pltpu
from jax.experimental.pallas import tpu_sc as plsc

N_NODES = 10000
D = 128
NC = 2
NS = 16
CHUNK = 128
N_PAD = 10240
HALF = N_PAD // NC
ACC_ROWS = HALF + 8
TRASH = HALF
ROWS_PER_TILE = HALF // NS
PAD_IDX = N_PAD - 1
NBUF = 2
BLK = 256
N_BLOCKS = N_PAD // BLK


def _sc_agg_body(nchunk, table, srcs, dsts, zinit, out,
                 src_v, dst_v, rows, gsems, ssems, acc):
    c = lax.axis_index("c")
    s = lax.axis_index("s")
    r0 = s * ROWS_PER_TILE
    pltpu.sync_copy(zinit, acc.at[pl.ds(r0, ROWS_PER_TILE)])
    pltpu.sync_copy(srcs.at[s], src_v)
    pltpu.sync_copy(dsts.at[c, s], dst_v)
    plsc.subcore_barrier()

    for b in range(NBUF):
        pltpu.async_copy(table.at[src_v.at[b]], rows.at[b], gsems.at[b])

    def body(j, carry):
        for b in range(NBUF):
            idx = NBUF * j + b
            pltpu.make_async_copy(table.at[src_v.at[idx]], rows.at[b],
                                  gsems.at[b]).wait()
            pltpu.async_copy(rows.at[b], acc.at[dst_v.at[idx]], ssems.at[b],
                             add=True)
            pltpu.make_async_copy(rows.at[b], acc.at[dst_v.at[idx]],
                                  ssems.at[b]).wait()
            pltpu.async_copy(table.at[src_v.at[idx + NBUF]], rows.at[b],
                             gsems.at[b])
        return carry

    def tail(j, carry):
        for b in range(NBUF):
            idx = NBUF * j + b
            pltpu.make_async_copy(table.at[src_v.at[idx]], rows.at[b],
                                  gsems.at[b]).wait()
            pltpu.sync_copy(rows.at[b], acc.at[dst_v.at[idx]], add=True)
        return carry

    ngroups = nchunk // NBUF
    lax.fori_loop(0, ngroups - 1, body, 0)
    tail(ngroups - 1, 0)
    plsc.subcore_barrier()
    pltpu.sync_copy(acc.at[pl.ds(r0, ROWS_PER_TILE)],
                    out.at[c, pl.ds(r0, ROWS_PER_TILE)])


def _sc_aggregate(table, srcs, dsts, zinit, d, nchunk):
    mesh = plsc.VectorSubcoreMesh(core_axis_name="c", subcore_axis_name="s",
                                  num_cores=NC, num_subcores=NS)
    kern = pl.kernel(
        functools.partial(_sc_agg_body, nchunk),
        out_type=jax.ShapeDtypeStruct((NC, HALF, d), jnp.float32),
        mesh=mesh,
        scratch_types=[
            pltpu.VMEM((nchunk, CHUNK), jnp.int32),
            pltpu.VMEM((nchunk, CHUNK), jnp.int32),
            pltpu.VMEM((NBUF, CHUNK, d), jnp.float32),
            pltpu.SemaphoreType.DMA((NBUF,)),
            pltpu.SemaphoreType.DMA((NBUF,)),
            pltpu.VMEM_SHARED((ACC_ROWS, d), jnp.float32),
        ],
        compiler_params=pltpu.CompilerParams(use_tc_tiling_on_sc=False),
        name=f"gcn_sc_agg_d{d}",
    )
    return kern(table, srcs, dsts, zinit)


def _tc_first_body(x_ref, w_ref, degp_ref, hs_ref, dinv_ref):
    i = pl.program_id(0)
    deg = jnp.sum(degp_ref[...], axis=1) + 1.0
    row = i * BLK + lax.broadcasted_iota(jnp.int32, (BLK,), 0)
    dinv = jnp.where(row < N_NODES, lax.rsqrt(deg), 0.0)
    dinv_b = jnp.broadcast_to(dinv[:, None], (BLK, D))
    dinv_ref[...] = dinv_b
    h = jnp.dot(x_ref[...], w_ref[...], preferred_element_type=jnp.float32)
    hs_ref[...] = h * dinv_b


def _tc_first(x_pad, w1, degs):
    return pl.pallas_call(
        _tc_first_body,
        grid=(N_BLOCKS,),
        in_specs=[
            pl.BlockSpec((BLK, D), lambda i: (i, 0)),
            pl.BlockSpec((D, D), lambda i: (0, 0)),
            pl.BlockSpec((BLK, 16), lambda i: (i, 0)),
        ],
        out_specs=[
            pl.BlockSpec((BLK, D), lambda i: (i, 0)),
            pl.BlockSpec((BLK, D), lambda i: (i, 0)),
        ],
        out_shape=[
            jax.ShapeDtypeStruct((N_PAD, D), jnp.float32),
            jax.ShapeDtypeStruct((N_PAD, D), jnp.float32),
        ],
        name="gcn_tc_first",
    )(x_pad, w1, degs)


def _tc_mid_body(agg_ref, hs_ref, dinv_ref, b_ref, w_ref, pre_ref, o_ref):
    tot = agg_ref[...] + hs_ref[...]
    pre = tot * dinv_ref[...] + b_ref[...]
    pre_ref[...] = pre
    act = jnp.maximum(pre, 0.0)
    o_ref[...] = jnp.dot(act, w_ref[...],
                         preferred_element_type=jnp.float32) * dinv_ref[...]


def _tc_mid(agg, hs, dinv_b, b, w_next):
    return pl.pallas_call(
        _tc_mid_body,
        grid=(N_BLOCKS,),
        in_specs=[
            pl.BlockSpec((BLK, D), lambda i: (i, 0)),
            pl.BlockSpec((BLK, D), lambda i: (i, 0)),
            pl.BlockSpec((BLK, D), lambda i: (i, 0)),
            pl.BlockSpec((1, D), lambda i: (0, 0)),
            pl.BlockSpec((D, D), lambda i: (0, 0)),
        ],
        out_specs=[
            pl.BlockSpec((BLK, D), lambda i: (i, 0)),
            pl.BlockSpec((BLK, D), lambda i: (i, 0)),
        ],
        out_shape=[
            jax.ShapeDtypeStruct((N_PAD, D), jnp.float32),
            jax.ShapeDtypeStruct((N_PAD, D), jnp.float32),
        ],
        name="gcn_tc_mid",
    )(agg, hs, dinv_b, b.reshape(1, D), w_next)


def kernel(x, edge_index, W1, b1, W2, b2, W3, b3):
    e = edge_index.shape[1]
    nchunk = -(-e // (NS * CHUNK))
    nchunk = -(-nchunk // NBUF) * NBUF
    e_pad = NS * nchunk * CHUNK
    src = edge_index[0].astype(jnp.int32)
    dst = edge_index[1].astype(jnp.int32)
    fill = jnp.full((e_pad - e,), PAD_IDX, jnp.int32)
    src = jnp.concatenate([src, fill])
    dst = jnp.concatenate([dst, fill])
    srcs = src.reshape(NS, nchunk, CHUNK)
    dst_loc = dst[None, :] - jnp.array([0, HALF], jnp.int32)[:, None]
    dsts = jnp.where((dst_loc >= 0) & (dst_loc < HALF), dst_loc, TRASH)
    dsts = dsts.reshape(NC, NS, nchunk, CHUNK)

    x_pad = jnp.pad(x, ((0, N_PAD - N_NODES), (0, 0)))
    e0_table = jnp.zeros((N_PAD, 16), jnp.float32).at[:N_NODES, 0].set(1.0)
    z16 = jnp.zeros((ROWS_PER_TILE, 16), jnp.float32)
    z128 = jnp.zeros((ROWS_PER_TILE, D), jnp.float32)

    degs = _sc_aggregate(e0_table, srcs, dsts, z16, 16, nchunk)
    hs1, dinv_b = _tc_first(x_pad, W1, degs.reshape(N_PAD, 16))

    def step(hs, wb):
        w_next, b = wb
        agg = _sc_aggregate(hs, srcs, dsts, z128, D, nchunk)
        pre, hs_next = _tc_mid(agg.reshape(N_PAD, D), hs, dinv_b, b, w_next)
        return hs_next, pre

    ws = jnp.stack([W2, W3, jnp.zeros_like(W3)])
    bs = jnp.stack([b1, b2, b3])
    _, pres = lax.scan(step, hs1, (ws, bs))
    return pres[2][:N_NODES]
